# Initial kernel scaffold; baseline (speedup 1.0000x reference)
import jax
import jax.numpy as jnp
from jax.experimental import pallas as pl


def kernel(item_id, cat_0, cat_1, cat_2, num_0, num_1, num_2, num_3, vec_0, item_table, cat_table_0, cat_table_1, cat_table_2, W_num1, b_num1, W_num2, b_num2, W_vec, b_vec, W_m1, b_m1, W_m2, b_m2):
    raise NotImplementedError("write your pallas kernel here")


# speedup vs baseline: 1.0252x; 1.0252x over previous
import functools

import jax
import jax.numpy as jnp
from jax import lax
from jax.experimental import pallas as pl
from jax.experimental.pallas import tpu as pltpu
from jax.experimental.pallas import tpu_sc as plsc

B = 16384
D = 32
NC = 2
NS = 16
NW = NC * NS
BPW = B // NW
CHUNK = 128
NCH = BPW // CHUNK

def _sc_gather4_body(idx0, idx1, idx2, idx3, tab0, tab1, tab2, tab3,
                     out0, out1, out2, out3,
                     iv0, iv1, iv2, iv3, rv0, rv1, rv2, rv3, sem):
    wid = lax.axis_index("s") * NC + lax.axis_index("c")
    base = wid * BPW
    idx_in = (idx0, idx1, idx2, idx3)
    tabs = (tab0, tab1, tab2, tab3)
    ivs = (iv0, iv1, iv2, iv3)
    rvs = (rv0, rv1, rv2, rv3)
    outs = (out0, out1, out2, out3)
    for t in range(4):
        pltpu.sync_copy(idx_in[t].at[wid], ivs[t])
    handles = []
    for t in range(4):
        for j in range(NCH):
            handles.append(
                pltpu.async_copy(tabs[t].at[ivs[t].at[j]],
                                 rvs[t].at[pl.ds(j * CHUNK, CHUNK)], sem))
    for h in handles:
        h.wait()
    for t in range(4):
        pltpu.sync_copy(rvs[t], outs[t].at[pl.ds(base, BPW)])


@functools.cache
def _sc_gather4_fn():
    mesh = plsc.VectorSubcoreMesh(
        core_axis_name="c", subcore_axis_name="s",
        num_cores=NC, num_subcores=NS)
    return pl.kernel(
        _sc_gather4_body,
        out_type=[jax.ShapeDtypeStruct((B, D), jnp.float32)] * 4,
        mesh=mesh,
        scratch_types=[
            pltpu.VMEM((NCH, CHUNK), jnp.int32),
            pltpu.VMEM((NCH, CHUNK), jnp.int32),
            pltpu.VMEM((NCH, CHUNK), jnp.int32),
            pltpu.VMEM((NCH, CHUNK), jnp.int32),
            pltpu.VMEM((BPW, D), jnp.float32),
            pltpu.VMEM((BPW, D), jnp.float32),
            pltpu.VMEM((BPW, D), jnp.float32),
            pltpu.VMEM((BPW, D), jnp.float32),
            pltpu.SemaphoreType.DMA,
        ],
        compiler_params=pltpu.CompilerParams(use_tc_tiling_on_sc=False),
    )


def _sc_gather4(*args):
    return _sc_gather4_fn()(*args)


_BLK = 2048
_GRID = B // _BLK


def _dense_body(num_ref, vec_ref, g0_ref, g1_ref, g2_ref, g3_ref,
                wn1_ref, bn1_ref, wn2_ref, bn2_ref, wv_ref, bv_ref,
                wm1_ref, bm1_ref, wm2_ref, bm2_ref, out_ref):
    f32 = jnp.float32
    h = jnp.dot(num_ref[...], wn1_ref[...], preferred_element_type=f32)
    h = jnp.maximum(h + bn1_ref[...], 0.0)
    h = jnp.dot(h, wn2_ref[...], preferred_element_type=f32) + bn2_ref[...]
    v = jnp.dot(vec_ref[...], wv_ref[...], preferred_element_type=f32) + bv_ref[...]
    wm1 = wm1_ref[...]
    x = jnp.dot(g0_ref[...], wm1[0:32], preferred_element_type=f32)
    x = x + jnp.dot(g1_ref[...], wm1[32:64], preferred_element_type=f32)
    x = x + jnp.dot(g2_ref[...], wm1[64:96], preferred_element_type=f32)
    x = x + jnp.dot(g3_ref[...], wm1[96:128], preferred_element_type=f32)
    x = x + jnp.dot(h, wm1[128:224], preferred_element_type=f32)
    x = x + jnp.dot(v, wm1[224:256], preferred_element_type=f32)
    x = jnp.maximum(x + bm1_ref[...], 0.0)
    out_ref[...] = jnp.dot(x, wm2_ref[...], preferred_element_type=f32) + bm2_ref[...]


def _full(shape):
    return pl.BlockSpec(shape, lambda i: (0, 0))


_dense = pl.pallas_call(
    _dense_body,
    grid=(_GRID,),
    in_specs=[
        pl.BlockSpec((_BLK, 4), lambda i: (i, 0)),
        pl.BlockSpec((_BLK, 128), lambda i: (i, 0)),
        pl.BlockSpec((_BLK, D), lambda i: (i, 0)),
        pl.BlockSpec((_BLK, D), lambda i: (i, 0)),
        pl.BlockSpec((_BLK, D), lambda i: (i, 0)),
        pl.BlockSpec((_BLK, D), lambda i: (i, 0)),
        _full((4, 96)), _full((1, 96)),
        _full((96, 96)), _full((1, 96)),
        _full((128, 32)), _full((1, 32)),
        _full((256, 128)), _full((1, 128)),
        _full((128, 32)), _full((1, 32)),
    ],
    out_specs=pl.BlockSpec((_BLK, D), lambda i: (i, 0)),
    out_shape=jax.ShapeDtypeStruct((B, D), jnp.float32),
    compiler_params=pltpu.CompilerParams(
        dimension_semantics=("arbitrary",),
    ),
)


def kernel(item_id, cat_0, cat_1, cat_2, num_0, num_1, num_2, num_3, vec_0,
           item_table, cat_table_0, cat_table_1, cat_table_2,
           W_num1, b_num1, W_num2, b_num2, W_vec, b_vec,
           W_m1, b_m1, W_m2, b_m2):
    def shape_idx(ix):
        return ix.astype(jnp.int32).reshape(NW, NCH, CHUNK)

    g_c0, g_c1, g_c2, g_it = _sc_gather4(
        shape_idx(cat_0), shape_idx(cat_1), shape_idx(cat_2), shape_idx(item_id),
        cat_table_0, cat_table_1, cat_table_2, item_table)

    numerical_v = jnp.concatenate([num_0, num_1, num_2, num_3], axis=1)
    return _dense(
        numerical_v, vec_0, g_c0, g_c1, g_c2, g_it,
        W_num1, b_num1.reshape(1, -1), W_num2, b_num2.reshape(1, -1),
        W_vec, b_vec.reshape(1, -1),
        W_m1, b_m1.reshape(1, -1), W_m2, b_m2.reshape(1, -1))


# speedup vs baseline: 2.0685x; 2.0176x over previous
import functools

import jax
import jax.numpy as jnp
from jax import lax
from jax.experimental import pallas as pl
from jax.experimental.pallas import tpu as pltpu
from jax.experimental.pallas import tpu_sc as plsc

B = 16384
D = 32
NC = 2
NS = 16
NW = NC * NS
BPW = B // NW
CHUNK = 128
NCH = BPW // CHUNK

BQ = 2048


def _ceil_div(a, b):
    return -(-a // b)


@functools.cache
def _pack_fn(v_rows):
    nb = _ceil_div(v_rows, 4 * BQ)
    max_blk = v_rows // BQ - 1

    def imap(i, k):
        b = 4 * i + k
        return (0, b if k == 0 else jnp.minimum(b, max_blk))

    def body(t0, t1, t2, t3, tout):
        f32 = jnp.float32
        row = lax.broadcasted_iota(jnp.int32, (32, 128), 0)
        col = lax.broadcasted_iota(jnp.int32, (32, 128), 1)
        dn = (((0,), (0,)), ((), ()))

        def tr(ref, k):
            ek = (col == row + 32 * k).astype(f32)
            return lax.dot_general(ref[...], ek, dn,
                                   preferred_element_type=f32)

        tout[...] = tr(t0, 0) + tr(t1, 1) + tr(t2, 2) + tr(t3, 3)

    return pl.pallas_call(
        body,
        grid=(nb,),
        in_specs=[
            pl.BlockSpec((32, BQ), lambda i, k=k: imap(i, k))
            for k in range(4)
        ],
        out_specs=pl.BlockSpec((BQ, 128), lambda i: (i, 0)),
        out_shape=jax.ShapeDtypeStruct((nb * BQ, 128), jnp.float32),
        compiler_params=pltpu.CompilerParams(
            fuse_transposed_lhs_in_matmul=True,
        ),
    )


def _pack(table):
    tt = table.T
    return _pack_fn(table.shape[0])(tt, tt, tt, tt)


def _sc_gather_body(idx0, idx1, idx2, idx3, tab0, tab1, tab2, tab3,
                    out, qv, pk, obuf, sem, sem2):
    wid = lax.axis_index("s") * NC + lax.axis_index("c")
    base = wid * BPW
    idxs = (idx0, idx1, idx2, idx3)
    tabs = (tab0, tab1, tab2, tab3)
    for t in range(4):
        pltpu.sync_copy(idxs[t].at[wid], qv)
        handles = []
        for c in range(NCH):
            handles.append(pltpu.async_copy(
                tabs[t].at[qv.at[c]],
                pk.at[t].at[pl.ds(c * CHUNK, CHUNK)], sem))
        for h in handles:
            h.wait()

    oh = {}
    for c in range(NCH):
        def step(i, _, c=c):
            for t in range(4):
                for half in range(2):
                    obuf[c % 2, i, pl.ds(t * 32 + half * 16, 16)] = (
                        pk[t, c * CHUNK + i, pl.ds(half * 16, 16)])
            return 0

        if c >= 2:
            oh[c - 2].wait()
        lax.fori_loop(0, CHUNK, step, 0)
        oh[c] = pltpu.async_copy(
            obuf.at[c % 2], out.at[pl.ds(base + c * CHUNK, CHUNK)], sem2)
    oh[NCH - 2].wait()
    oh[NCH - 1].wait()


@functools.cache
def _sc_gather_fn(vp0, vp1, vp2, vp3):
    mesh = plsc.VectorSubcoreMesh(
        core_axis_name="c", subcore_axis_name="s",
        num_cores=NC, num_subcores=NS)
    return pl.kernel(
        _sc_gather_body,
        out_type=jax.ShapeDtypeStruct((B, 128), jnp.float32),
        mesh=mesh,
        scratch_types=[
            pltpu.VMEM((NCH, CHUNK), jnp.int32),
            pltpu.VMEM((4, BPW, D), jnp.float32),
            pltpu.VMEM((2, CHUNK, 128), jnp.float32),
            pltpu.SemaphoreType.DMA,
            pltpu.SemaphoreType.DMA,
        ],
        compiler_params=pltpu.CompilerParams(use_tc_tiling_on_sc=False),
    )


_BLK = 2048
_GRID = B // _BLK


def _dense_body(num_ref, vec_ref, pk_ref,
                wn1_ref, bn1_ref, wn2_ref, bn2_ref, wv_ref, bv_ref,
                wm1_ref, bm1_ref, wm2_ref, bm2_ref, out_ref):
    f32 = jnp.float32
    h = jnp.dot(num_ref[...], wn1_ref[...], preferred_element_type=f32)
    h = jnp.maximum(h + bn1_ref[...], 0.0)
    h = jnp.dot(h, wn2_ref[...], preferred_element_type=f32) + bn2_ref[...]
    v = jnp.dot(vec_ref[...], wv_ref[...], preferred_element_type=f32) + bv_ref[...]
    wm1 = wm1_ref[...]
    x = jnp.dot(pk_ref[...], wm1[0:128], preferred_element_type=f32)
    x = x + jnp.dot(h, wm1[128:224], preferred_element_type=f32)
    x = x + jnp.dot(v, wm1[224:256], preferred_element_type=f32)
    x = jnp.maximum(x + bm1_ref[...], 0.0)
    out_ref[...] = jnp.dot(x, wm2_ref[...], preferred_element_type=f32) + bm2_ref[...]


def _full(shape):
    return pl.BlockSpec(shape, lambda i: (0, 0))


_dense = pl.pallas_call(
    _dense_body,
    grid=(_GRID,),
    in_specs=[
        pl.BlockSpec((_BLK, 4), lambda i: (i, 0)),
        pl.BlockSpec((_BLK, 128), lambda i: (i, 0)),
        pl.BlockSpec((_BLK, 128), lambda i: (i, 0)),
        _full((4, 96)), _full((1, 96)),
        _full((96, 96)), _full((1, 96)),
        _full((128, 32)), _full((1, 32)),
        _full((256, 128)), _full((1, 128)),
        _full((128, 32)), _full((1, 32)),
    ],
    out_specs=pl.BlockSpec((_BLK, D), lambda i: (i, 0)),
    out_shape=jax.ShapeDtypeStruct((B, D), jnp.float32),
    compiler_params=pltpu.CompilerParams(
        dimension_semantics=("arbitrary",),
    ),
)


def kernel(item_id, cat_0, cat_1, cat_2, num_0, num_1, num_2, num_3, vec_0,
           item_table, cat_table_0, cat_table_1, cat_table_2,
           W_num1, b_num1, W_num2, b_num2, W_vec, b_vec,
           W_m1, b_m1, W_m2, b_m2):
    p0 = _pack(cat_table_0).reshape(-1, D)
    p1 = _pack(cat_table_1).reshape(-1, D)
    p2 = _pack(cat_table_2).reshape(-1, D)
    p3 = _pack(item_table).reshape(-1, D)

    def gidx(ix):
        r = ix.astype(jnp.int32)
        g = ((r >> 13) << 13) + ((r & 2047) << 2) + ((r >> 11) & 3)
        return g.reshape(NW, NCH, CHUNK)

    gather = _sc_gather_fn(p0.shape[0], p1.shape[0], p2.shape[0], p3.shape[0])
    pk = gather(gidx(cat_0), gidx(cat_1), gidx(cat_2), gidx(item_id),
                p0, p1, p2, p3)

    numerical_v = jnp.concatenate([num_0, num_1, num_2, num_3], axis=1)
    return _dense(
        numerical_v, vec_0, pk,
        W_num1, b_num1.reshape(1, -1), W_num2, b_num2.reshape(1, -1),
        W_vec, b_vec.reshape(1, -1),
        W_m1, b_m1.reshape(1, -1), W_m2, b_m2.reshape(1, -1))
